# baseline (speedup 1.0000x reference)
import functools

import jax
import jax.numpy as jnp
from jax import lax
from jax.experimental import pallas as pl
from jax.experimental.pallas import tpu as pltpu
from jax.experimental.pallas import tpu_sc as plsc

_POSITION = 0


@functools.partial(jax.jit, static_argnums=(2, 3))
def _pool_rows(table, idx, b, d):
    mesh = plsc.VectorSubcoreMesh(core_axis_name="c", subcore_axis_name="s")

    @functools.partial(
        pl.kernel,
        out_type=jax.ShapeDtypeStruct((b, d), jnp.float32),
        mesh=mesh,
        scratch_types=[
            pltpu.VMEM((b,), jnp.int32),
            pltpu.VMEM((b, d), jnp.float32),
            pltpu.SemaphoreType.DMA,
        ],
    )
    def pooler(table_hbm, idx_hbm, out_hbm, idx_v, rows_v, sem):
        cid = lax.axis_index("c")
        sid = lax.axis_index("s")

        @pl.when(jnp.logical_and(cid == 0, sid == 0))
        def _():
            pltpu.sync_copy(idx_hbm, idx_v)
            pltpu.async_copy(table_hbm.at[idx_v], rows_v, sem).wait()
            pltpu.sync_copy(rows_v, out_hbm)

    return pooler(table, idx)


def kernel(inputs, mask):
    del mask
    b, s, d = inputs.shape
    table = inputs.reshape(b * s, d)
    idx = jnp.arange(b, dtype=jnp.int32) * s + _POSITION
    return _pool_rows(table, idx, b, d)


# speedup vs baseline: 1.0975x; 1.0975x over previous
import functools

import jax
import jax.numpy as jnp
from jax import lax
from jax.experimental import pallas as pl
from jax.experimental.pallas import tpu as pltpu
from jax.experimental.pallas import tpu_sc as plsc

_POSITION = 0


@functools.partial(jax.jit, static_argnums=(1,))
def _pool_rows(inputs, position):
    b, _, d = inputs.shape
    mesh = plsc.ScalarSubcoreMesh(axis_name="c", num_cores=2)

    @functools.partial(
        pl.kernel,
        out_type=jax.ShapeDtypeStruct((b, d), jnp.float32),
        mesh=mesh,
        scratch_types=[pltpu.SemaphoreType.DMA],
    )
    def pooler(in_hbm, out_hbm, sem):
        @pl.when(lax.axis_index("c") == 0)
        def _():
            pltpu.async_copy(in_hbm.at[:, position], out_hbm, sem).wait()

    return pooler(inputs)


def kernel(inputs, mask):
    del mask
    return _pool_rows(inputs, _POSITION)


# speedup vs baseline: 1.1772x; 1.0726x over previous
import functools

import jax
import jax.numpy as jnp
from jax import lax
from jax.experimental import pallas as pl
from jax.experimental.pallas import tpu as pltpu
from jax.experimental.pallas import tpu_sc as plsc

_POSITION = 0


@functools.partial(jax.jit, static_argnums=(1,))
def _pool_rows(inputs, position):
    b, _, d = inputs.shape
    mesh = plsc.ScalarSubcoreMesh(axis_name="c", num_cores=1)

    @functools.partial(
        pl.kernel,
        out_type=jax.ShapeDtypeStruct((b, d), jnp.float32),
        mesh=mesh,
        scratch_types=[pltpu.SemaphoreType.DMA],
    )
    def pooler(in_hbm, out_hbm, sem):
        @pl.when(lax.axis_index("c") == 0)
        def _():
            pltpu.async_copy(in_hbm.at[:, position], out_hbm, sem).wait()

    return pooler(inputs)


def kernel(inputs, mask):
    del mask
    return _pool_rows(inputs, _POSITION)


# speedup vs baseline: 1.1815x; 1.0036x over previous
import functools

import jax
import jax.numpy as jnp
from jax import lax
from jax.experimental import pallas as pl
from jax.experimental.pallas import tpu as pltpu
from jax.experimental.pallas import tpu_sc as plsc

_POSITION = 0


@functools.partial(jax.jit, static_argnums=(1,))
def _pool_rows(inputs, position):
    b, _, d = inputs.shape
    mesh = plsc.ScalarSubcoreMesh(axis_name="c", num_cores=1)

    @functools.partial(
        pl.kernel,
        out_type=jax.ShapeDtypeStruct((b, d), jnp.float32),
        mesh=mesh,
        scratch_types=[pltpu.SemaphoreType.DMA],
    )
    def pooler(in_hbm, out_hbm, sem):
        copies = [
            pltpu.async_copy(in_hbm.at[i, position], out_hbm.at[i], sem)
            for i in range(b)
        ]
        for c in copies:
            c.wait()

    return pooler(inputs)


def kernel(inputs, mask):
    del mask
    return _pool_rows(inputs, _POSITION)


# speedup vs baseline: 1.1875x; 1.0051x over previous
import functools

import jax
import jax.numpy as jnp
from jax import lax
from jax.experimental import pallas as pl
from jax.experimental.pallas import tpu as pltpu
from jax.experimental.pallas import tpu_sc as plsc

_POSITION = 0


@functools.partial(jax.jit, static_argnums=(1,))
def _pool_rows(inputs, position):
    b, _, d = inputs.shape
    mesh = plsc.ScalarSubcoreMesh(axis_name="c", num_cores=1)

    @functools.partial(
        pl.kernel,
        out_type=jax.ShapeDtypeStruct((b, d), jnp.float32),
        mesh=mesh,
        scratch_types=[pltpu.SemaphoreType.DMA],
    )
    def pooler(in_hbm, out_hbm, sem):
        pltpu.async_copy(in_hbm.at[:, position], out_hbm, sem).wait()

    return pooler(inputs)


def kernel(inputs, mask):
    del mask
    return _pool_rows(inputs, _POSITION)
